# baseline (speedup 1.0000x reference)
import functools

import jax
import jax.numpy as jnp
from jax import lax
from jax.experimental import pallas as pl
from jax.experimental.pallas import tpu as pltpu
from jax.experimental.pallas import tpu_sc as plsc

_NC = 2
_NS = 16
_L = 16
_NW = _NC * _NS


def _prep(UT, VTp, vposT, vnegT, B, W, N):
    D, VOC = UT.shape
    VP = VTp.shape[1]
    H = VP // 2
    VR = ((VOC + 127) // 128) * 128
    Q = B // _NW // 128

    H4 = VP // 4
    JR = (W + N) * Q

    def body(u_ref, v_ref, p_ref, n_ref, g_ref, idx_ref):
        x = lax.dot_general(u_ref[...], v_ref[...],
                            (((0,), (0,)), ((), ())),
                            preferred_element_type=jnp.float32)
        bs = [lax.bitcast_convert_type(
                  x[:, i * H4:(i + 1) * H4].astype(jnp.float8_e4m3fn),
                  jnp.uint8).astype(jnp.uint32) for i in range(4)]
        packed = (bs[0] | (bs[1] << 8) | (bs[2] << 16)
                  | (bs[3] << 24)).astype(jnp.int32)
        g_ref[...] = jnp.concatenate(
            [packed, jnp.zeros((VR - VOC, H4), jnp.int32)], axis=0)
        n = n_ref[...].reshape(N, _NW, Q, 128)
        n = n.transpose(1, 0, 2, 3).reshape(_NW, N * Q, 128)
        p = p_ref[...].reshape(W, _NW, Q, 128)
        p = p.transpose(1, 0, 2, 3).reshape(_NW, W * Q, 128)
        a = jnp.concatenate([n, p], axis=1)
        idx_ref[...] = a[:, :JR // 2, :] | (a[:, JR // 2:, :] << 16)

    return pl.pallas_call(
        body,
        out_shape=(jax.ShapeDtypeStruct((VR, H4), jnp.int32),
                   jax.ShapeDtypeStruct((_NW, JR // 2, 128), jnp.int32)),
    )(UT, VTp, vposT, vnegT)


def _sc_scores(G, u_idx, vidxT, B, W, N):
    VR, H4 = G.shape
    GPW = B // _L // _NW
    BPW = GPW * _L
    GPC = 4
    CL = GPC * _L
    NCH = GPW // GPC
    JH = (W + N) // 2
    JRH = (W + N) * (BPW // 128) // 2
    ASH = H4.bit_length() - 4

    mesh = plsc.VectorSubcoreMesh(core_axis_name="c", subcore_axis_name="s")

    @functools.partial(
        pl.kernel,
        out_type=(jax.ShapeDtypeStruct((B,), jnp.float32),
                  jax.ShapeDtypeStruct((B,), jnp.float32)),
        mesh=mesh,
        scratch_types=[
            pltpu.VMEM((JRH, 128), jnp.int32),
            pltpu.VMEM((BPW,), jnp.int32),
            pltpu.VMEM((2 * CL, H4), jnp.int32),
            pltpu.VMEM((BPW,), jnp.float32),
            pltpu.VMEM((BPW,), jnp.float32),
            pltpu.SemaphoreType.DMA,
            pltpu.SemaphoreType.DMA,
        ],
        compiler_params=pltpu.CompilerParams(use_tc_tiling_on_sc=False,
                                             needs_layout_passes=False),
    )
    def sck(g_hbm, u_hbm, vidx_hbm, pos_hbm, neg_hbm,
            vidx_v, u_v, rows_v, pos_v, neg_v, sem0, sem1):
        cid = lax.axis_index("c")
        sid = lax.axis_index("s")
        wid = sid * _NC + cid
        b0 = wid * BPW
        pltpu.sync_copy(vidx_hbm.at[wid], vidx_v)
        pltpu.sync_copy(u_hbm.at[pl.ds(b0, BPW)], u_v)

        sems = (sem0, sem1)
        iota = lax.broadcasted_iota(jnp.int32, (_L,), 0)

        def rows_dma(c, p, q):
            return pltpu.make_async_copy(
                g_hbm.at[u_v.at[pl.ds(c * CL + q * _L, _L)]],
                rows_v.at[pl.ds(p * CL + q * _L, _L)],
                sems[p])

        def start_chunk(c, p):
            for q in range(GPC):
                rows_dma(c, p, q).start()

        def wait_chunk(c, p):
            for q in range(GPC):
                rows_dma(c, p, q).wait()

        def chunk(c, p):
            @pl.when(c + 1 < NCH)
            def _():
                start_chunk(c + 1, 1 - p)
            wait_chunk(c, p)
            scale = jnp.float32(2.0 ** 120)
            for k in range(GPC):
                g = c * GPC + k
                row_base = p * CL + k * _L + iota
                jrow = g // 8
                jcol = (g % 8) * _L

                def decode(v):
                    w = plsc.load_gather(rows_v, [row_base,
                                                  jnp.bitwise_and(v, H4 - 1)])
                    amt = lax.shift_right_logical(
                        jnp.bitwise_and(v, 3 * H4), ASH)
                    b = jnp.bitwise_and(lax.shift_right_logical(w, amt), 0xFF)
                    bits = (lax.shift_left(jnp.bitwise_and(b, 0x80), 24)
                            | lax.shift_left(jnp.bitwise_and(b, 0x7F), 20))
                    return plsc.bitcast(bits, jnp.float32) * scale

                neg = None
                pos = None
                for jp in range(JH):
                    wv = vidx_v[jp * 4 + jrow, pl.ds(jcol, _L)]
                    flo = decode(jnp.bitwise_and(wv, 0xFFFF))
                    fhi = decode(lax.shift_right_logical(wv, 16))
                    neg = flo if neg is None else neg + flo
                    if jp < N - JH:
                        neg = neg + fhi
                    elif pos is None:
                        pos = fhi
                    else:
                        pos = pos + fhi
                pos_v[pl.ds(g * _L, _L)] = pos
                neg_v[pl.ds(g * _L, _L)] = neg

        start_chunk(0, 0)

        def lbody(i, carry):
            chunk(2 * i, 0)
            chunk(2 * i + 1, 1)
            return carry

        lax.fori_loop(0, NCH // 2, lbody, 0)
        pltpu.sync_copy(pos_v, pos_hbm.at[pl.ds(b0, BPW)])
        pltpu.sync_copy(neg_v, neg_hbm.at[pl.ds(b0, BPW)])

    return sck(G, u_idx, vidxT)


def _loss(pos2d, neg2d, B):
    def body(p_ref, n_ref, o_ref):
        p = p_ref[...]
        n = n_ref[...]
        t = jax.nn.log_sigmoid(p) + jax.nn.log_sigmoid(-n)
        o_ref[...] = -jnp.sum(t, keepdims=True).reshape(1, 1) / B

    return pl.pallas_call(
        body,
        out_shape=jax.ShapeDtypeStruct((1, 1), jnp.float32),
    )(pos2d, neg2d)


def kernel(u_pos, v_pos, v_neg, batch_size, U_emb, V_emb):
    B = u_pos.shape[0]
    W = v_pos.shape[1]
    N = v_neg.shape[1]
    VOC, D = U_emb.shape

    VP = ((VOC + 127) // 128) * 128
    VTp = jnp.pad(jnp.transpose(V_emb), ((0, 0), (0, VP - VOC)))
    G, vidxT = _prep(jnp.transpose(U_emb), VTp,
                     jnp.transpose(v_pos).astype(jnp.int32),
                     jnp.transpose(v_neg).astype(jnp.int32), B, W, N)
    u_idx = u_pos.reshape(B).astype(jnp.int32)

    pos_s, neg_s = _sc_scores(G, u_idx, vidxT, B, W, N)
    out = _loss(pos_s.reshape(128, B // 128), neg_s.reshape(128, B // 128), B)
    return out[0, 0]


# speedup vs baseline: 1.0141x; 1.0141x over previous
import functools

import jax
import jax.numpy as jnp
from jax import lax
from jax.experimental import pallas as pl
from jax.experimental.pallas import tpu as pltpu
from jax.experimental.pallas import tpu_sc as plsc

_NC = 2
_NS = 16
_L = 16
_NW = _NC * _NS


def _prep(UT, VTp, vposT, vnegT, B, W, N):
    D, VOC = UT.shape
    VP = VTp.shape[1]
    H = VP // 2
    VR = ((VOC + 127) // 128) * 128
    Q = B // _NW // 128

    JR = (W + N) * Q

    def body(u_ref, v_ref, p_ref, n_ref, g_ref, idx_ref):
        x = lax.dot_general(u_ref[...], v_ref[...],
                            (((0,), (0,)), ((), ())),
                            preferred_element_type=jnp.float32)
        g_ref[...] = jnp.concatenate(
            [x, jnp.zeros((VR - VOC, VP), jnp.float32)], axis=0)
        n = n_ref[...].reshape(N, _NW, Q, 128)
        n = n.transpose(1, 0, 2, 3).reshape(_NW, N * Q, 128)
        p = p_ref[...].reshape(W, _NW, Q, 128)
        p = p.transpose(1, 0, 2, 3).reshape(_NW, W * Q, 128)
        a = jnp.concatenate([n, p], axis=1)
        idx_ref[...] = a[:, :JR // 2, :] | (a[:, JR // 2:, :] << 16)

    return pl.pallas_call(
        body,
        out_shape=(jax.ShapeDtypeStruct((VR, VP), jnp.float32),
                   jax.ShapeDtypeStruct((_NW, JR // 2, 128), jnp.int32)),
    )(UT, VTp, vposT, vnegT)


def _sc_scores(G, u_idx, vidxT, B, W, N):
    VR, VP = G.shape
    GPW = B // _L // _NW
    BPW = GPW * _L
    GPC = 2
    CL = GPC * _L
    NCH = GPW // GPC
    JH = (W + N) // 2
    JRH = (W + N) * (BPW // 128) // 2

    mesh = plsc.VectorSubcoreMesh(core_axis_name="c", subcore_axis_name="s")

    @functools.partial(
        pl.kernel,
        out_type=(jax.ShapeDtypeStruct((B,), jnp.float32),
                  jax.ShapeDtypeStruct((B,), jnp.float32)),
        mesh=mesh,
        scratch_types=[
            pltpu.VMEM((JRH, 128), jnp.int32),
            pltpu.VMEM((BPW,), jnp.int32),
            pltpu.VMEM((2 * CL, VP), jnp.float32),
            pltpu.VMEM((BPW,), jnp.float32),
            pltpu.VMEM((BPW,), jnp.float32),
            pltpu.SemaphoreType.DMA,
            pltpu.SemaphoreType.DMA,
        ],
        compiler_params=pltpu.CompilerParams(use_tc_tiling_on_sc=False,
                                             needs_layout_passes=False),
    )
    def sck(g_hbm, u_hbm, vidx_hbm, pos_hbm, neg_hbm,
            vidx_v, u_v, rows_v, pos_v, neg_v, sem0, sem1):
        cid = lax.axis_index("c")
        sid = lax.axis_index("s")
        wid = sid * _NC + cid
        b0 = wid * BPW
        pltpu.sync_copy(vidx_hbm.at[wid], vidx_v)
        pltpu.sync_copy(u_hbm.at[pl.ds(b0, BPW)], u_v)

        sems = (sem0, sem1)
        iota = lax.broadcasted_iota(jnp.int32, (_L,), 0)

        def rows_dma(c, p, q):
            return pltpu.make_async_copy(
                g_hbm.at[u_v.at[pl.ds(c * CL + q * _L, _L)]],
                rows_v.at[pl.ds(p * CL + q * _L, _L)],
                sems[p])

        def start_chunk(c, p):
            for q in range(GPC):
                rows_dma(c, p, q).start()

        def wait_chunk(c, p):
            for q in range(GPC):
                rows_dma(c, p, q).wait()

        def chunk(c, p):
            @pl.when(c + 1 < NCH)
            def _():
                start_chunk(c + 1, 1 - p)
            wait_chunk(c, p)
            for k in range(GPC):
                g = c * GPC + k
                row_base = p * CL + k * _L + iota
                jrow = g // 8
                jcol = (g % 8) * _L

                def decode(v):
                    return plsc.load_gather(rows_v, [row_base, v])

                neg = None
                pos = None
                for jp in range(JH):
                    wv = vidx_v[jp * 4 + jrow, pl.ds(jcol, _L)]
                    flo = decode(jnp.bitwise_and(wv, 0xFFFF))
                    fhi = decode(lax.shift_right_logical(wv, 16))
                    neg = flo if neg is None else neg + flo
                    if jp < N - JH:
                        neg = neg + fhi
                    elif pos is None:
                        pos = fhi
                    else:
                        pos = pos + fhi
                pos_v[pl.ds(g * _L, _L)] = pos
                neg_v[pl.ds(g * _L, _L)] = neg

        start_chunk(0, 0)

        def lbody(i, carry):
            chunk(2 * i, 0)
            chunk(2 * i + 1, 1)
            return carry

        lax.fori_loop(0, NCH // 2, lbody, 0)
        pltpu.sync_copy(pos_v, pos_hbm.at[pl.ds(b0, BPW)])
        pltpu.sync_copy(neg_v, neg_hbm.at[pl.ds(b0, BPW)])

    return sck(G, u_idx, vidxT)


def _loss(pos2d, neg2d, B):
    def body(p_ref, n_ref, o_ref):
        p = p_ref[...]
        n = n_ref[...]
        t = jax.nn.log_sigmoid(p) + jax.nn.log_sigmoid(-n)
        o_ref[...] = -jnp.sum(t, keepdims=True).reshape(1, 1) / B

    return pl.pallas_call(
        body,
        out_shape=jax.ShapeDtypeStruct((1, 1), jnp.float32),
    )(pos2d, neg2d)


def kernel(u_pos, v_pos, v_neg, batch_size, U_emb, V_emb):
    B = u_pos.shape[0]
    W = v_pos.shape[1]
    N = v_neg.shape[1]
    VOC, D = U_emb.shape

    VP = ((VOC + 127) // 128) * 128
    VTp = jnp.pad(jnp.transpose(V_emb), ((0, 0), (0, VP - VOC)))
    G, vidxT = _prep(jnp.transpose(U_emb), VTp,
                     jnp.transpose(v_pos).astype(jnp.int32),
                     jnp.transpose(v_neg).astype(jnp.int32), B, W, N)
    u_idx = u_pos.reshape(B).astype(jnp.int32)

    pos_s, neg_s = _sc_scores(G, u_idx, vidxT, B, W, N)
    out = _loss(pos_s.reshape(128, B // 128), neg_s.reshape(128, B // 128), B)
    return out[0, 0]


# speedup vs baseline: 1.1653x; 1.1491x over previous
import functools

import jax
import jax.numpy as jnp
from jax import lax
from jax.experimental import pallas as pl
from jax.experimental.pallas import tpu as pltpu
from jax.experimental.pallas import tpu_sc as plsc

_NC = 2
_NS = 16
_L = 16
_NW = _NC * _NS


def _prep(UT, VTp, vposT, vnegT, B, W, N):
    D, VOC = UT.shape
    VP = VTp.shape[1]
    H = VP // 2
    VR = ((VOC + 127) // 128) * 128
    Q = B // _NW // 128

    JR = (W + N) * Q

    def body(u_ref, v_ref, p_ref, n_ref, g_ref, idx_ref):
        x = lax.dot_general(u_ref[...], v_ref[...],
                            (((0,), (0,)), ((), ())),
                            preferred_element_type=jnp.float32)
        lo = lax.bitcast_convert_type(
            x[:, :H].astype(jnp.bfloat16), jnp.uint16).astype(jnp.uint32)
        hi = lax.bitcast_convert_type(
            x[:, H:].astype(jnp.bfloat16), jnp.uint16).astype(jnp.uint32)
        packed = (lo | (hi << 16)).astype(jnp.int32)
        g_ref[...] = jnp.concatenate(
            [packed, jnp.zeros((VR - VOC, H), jnp.int32)], axis=0)
        n = n_ref[...].reshape(N, _NW, Q, 128)
        n = n.transpose(1, 0, 2, 3).reshape(_NW, N * Q, 128)
        p = p_ref[...].reshape(W, _NW, Q, 128)
        p = p.transpose(1, 0, 2, 3).reshape(_NW, W * Q, 128)
        a = jnp.concatenate([n, p], axis=1)
        idx_ref[...] = a[:, :JR // 2, :] | (a[:, JR // 2:, :] << 16)

    return pl.pallas_call(
        body,
        out_shape=(jax.ShapeDtypeStruct((VR, H), jnp.int32),
                   jax.ShapeDtypeStruct((_NW, JR // 2, 128), jnp.int32)),
    )(UT, VTp, vposT, vnegT)


def _sc_scores(G, u_idx, vidxT, B, W, N):
    VR, H = G.shape
    GPW = B // _L // _NW
    BPW = GPW * _L
    GPC = 4
    CL = GPC * _L
    NCH = GPW // GPC
    JH = (W + N) // 2
    JRH = (W + N) * (BPW // 128) // 2

    mesh = plsc.VectorSubcoreMesh(core_axis_name="c", subcore_axis_name="s")

    @functools.partial(
        pl.kernel,
        out_type=(jax.ShapeDtypeStruct((B,), jnp.float32),
                  jax.ShapeDtypeStruct((B,), jnp.float32)),
        mesh=mesh,
        scratch_types=[
            pltpu.VMEM((JRH, 128), jnp.int32),
            pltpu.VMEM((BPW,), jnp.int32),
            pltpu.VMEM((2 * CL, H), jnp.int32),
            pltpu.VMEM((BPW,), jnp.float32),
            pltpu.VMEM((BPW,), jnp.float32),
            pltpu.SemaphoreType.DMA,
            pltpu.SemaphoreType.DMA,
        ],
        compiler_params=pltpu.CompilerParams(use_tc_tiling_on_sc=False,
                                             needs_layout_passes=False),
    )
    def sck(g_hbm, u_hbm, vidx_hbm, pos_hbm, neg_hbm,
            vidx_v, u_v, rows_v, pos_v, neg_v, sem0, sem1):
        cid = lax.axis_index("c")
        sid = lax.axis_index("s")
        wid = sid * _NC + cid
        b0 = wid * BPW
        pltpu.sync_copy(vidx_hbm.at[wid], vidx_v)
        pltpu.sync_copy(u_hbm.at[pl.ds(b0, BPW)], u_v)

        sems = (sem0, sem1)
        iota = lax.broadcasted_iota(jnp.int32, (_L,), 0)

        def rows_dma(c, p, q):
            return pltpu.make_async_copy(
                g_hbm.at[u_v.at[pl.ds(c * CL + q * _L, _L)]],
                rows_v.at[pl.ds(p * CL + q * _L, _L)],
                sems[p])

        def start_chunk(c, p):
            for q in range(GPC):
                rows_dma(c, p, q).start()

        def wait_chunk(c, p):
            for q in range(GPC):
                rows_dma(c, p, q).wait()

        def chunk(c, p):
            @pl.when(c + 1 < NCH)
            def _():
                start_chunk(c + 1, 1 - p)
            wait_chunk(c, p)
            for k in range(GPC):
                g = c * GPC + k
                row_base = p * CL + k * _L + iota
                jrow = g // 8
                jcol = (g % 8) * _L

                def decode(v):
                    w = plsc.load_gather(rows_v, [row_base,
                                                  jnp.bitwise_and(v, H - 1)])
                    amt = lax.shift_right_logical(jnp.bitwise_and(v, H), 5)
                    bits = lax.shift_left(lax.shift_right_logical(w, amt), 16)
                    return plsc.bitcast(bits, jnp.float32)

                neg = None
                pos = None
                for jp in range(JH):
                    wv = vidx_v[jp * 4 + jrow, pl.ds(jcol, _L)]
                    flo = decode(jnp.bitwise_and(wv, 0xFFFF))
                    fhi = decode(lax.shift_right_logical(wv, 16))
                    neg = flo if neg is None else neg + flo
                    if jp < N - JH:
                        neg = neg + fhi
                    elif pos is None:
                        pos = fhi
                    else:
                        pos = pos + fhi
                pos_v[pl.ds(g * _L, _L)] = pos
                neg_v[pl.ds(g * _L, _L)] = neg

        start_chunk(0, 0)

        def lbody(i, carry):
            chunk(2 * i, 0)
            chunk(2 * i + 1, 1)
            return carry

        lax.fori_loop(0, NCH // 2, lbody, 0)
        pltpu.sync_copy(pos_v, pos_hbm.at[pl.ds(b0, BPW)])
        pltpu.sync_copy(neg_v, neg_hbm.at[pl.ds(b0, BPW)])

    return sck(G, u_idx, vidxT)


def _loss(pos2d, neg2d, B):
    def body(p_ref, n_ref, o_ref):
        p = p_ref[...]
        n = n_ref[...]
        t = jax.nn.log_sigmoid(p) + jax.nn.log_sigmoid(-n)
        o_ref[...] = -jnp.sum(t, keepdims=True).reshape(1, 1) / B

    return pl.pallas_call(
        body,
        out_shape=jax.ShapeDtypeStruct((1, 1), jnp.float32),
    )(pos2d, neg2d)


def kernel(u_pos, v_pos, v_neg, batch_size, U_emb, V_emb):
    B = u_pos.shape[0]
    W = v_pos.shape[1]
    N = v_neg.shape[1]
    VOC, D = U_emb.shape

    VP = ((VOC + 127) // 128) * 128
    VTp = jnp.pad(jnp.transpose(V_emb), ((0, 0), (0, VP - VOC)))
    G, vidxT = _prep(jnp.transpose(U_emb), VTp,
                     jnp.transpose(v_pos).astype(jnp.int32),
                     jnp.transpose(v_neg).astype(jnp.int32), B, W, N)
    u_idx = u_pos.reshape(B).astype(jnp.int32)

    pos_s, neg_s = _sc_scores(G, u_idx, vidxT, B, W, N)
    out = _loss(pos_s.reshape(128, B // 128), neg_s.reshape(128, B // 128), B)
    return out[0, 0]


# speedup vs baseline: 1.1704x; 1.0044x over previous
import functools

import jax
import jax.numpy as jnp
from jax import lax
from jax.experimental import pallas as pl
from jax.experimental.pallas import tpu as pltpu
from jax.experimental.pallas import tpu_sc as plsc

_NC = 2
_NS = 16
_L = 16
_NW = _NC * _NS


def _prep(UT, VTp, vposT, vnegT, B, W, N):
    D, VOC = UT.shape
    VP = VTp.shape[1]
    H = VP // 2
    VR = ((VOC + 127) // 128) * 128
    Q = B // _NW // 128

    JR = (W + N) * Q

    def body(u_ref, v_ref, p_ref, n_ref, g_ref, idx_ref):
        x = lax.dot_general(u_ref[...], v_ref[...],
                            (((0,), (0,)), ((), ())),
                            preferred_element_type=jnp.float32)
        lo = lax.bitcast_convert_type(
            x[:, :H].astype(jnp.bfloat16), jnp.uint16).astype(jnp.uint32)
        hi = lax.bitcast_convert_type(
            x[:, H:].astype(jnp.bfloat16), jnp.uint16).astype(jnp.uint32)
        packed = (lo | (hi << 16)).astype(jnp.int32)
        g_ref[...] = jnp.concatenate(
            [packed, jnp.zeros((VR - VOC, H), jnp.int32)], axis=0)
        JH = (W + N) // 2
        n = n_ref[...]
        hi = jnp.concatenate([n[JH:, :], p_ref[...]], axis=0)
        a = n[:JH, :] | (hi << 16)
        a = a.reshape(JH, _NW, Q, 128)
        idx_ref[...] = a.transpose(1, 0, 2, 3).reshape(_NW, JH * Q, 128)

    return pl.pallas_call(
        body,
        out_shape=(jax.ShapeDtypeStruct((VR, H), jnp.int32),
                   jax.ShapeDtypeStruct((_NW, JR // 2, 128), jnp.int32)),
    )(UT, VTp, vposT, vnegT)


def _sc_scores(G, u_idx, vidxT, B, W, N):
    VR, H = G.shape
    GPW = B // _L // _NW
    BPW = GPW * _L
    GPC = 4
    CL = GPC * _L
    NCH = GPW // GPC
    JH = (W + N) // 2
    JRH = (W + N) * (BPW // 128) // 2

    mesh = plsc.VectorSubcoreMesh(core_axis_name="c", subcore_axis_name="s")

    @functools.partial(
        pl.kernel,
        out_type=(jax.ShapeDtypeStruct((B,), jnp.float32),
                  jax.ShapeDtypeStruct((B,), jnp.float32)),
        mesh=mesh,
        scratch_types=[
            pltpu.VMEM((JRH, 128), jnp.int32),
            pltpu.VMEM((BPW,), jnp.int32),
            pltpu.VMEM((2 * CL, H), jnp.int32),
            pltpu.VMEM((BPW,), jnp.float32),
            pltpu.VMEM((BPW,), jnp.float32),
            pltpu.SemaphoreType.DMA,
            pltpu.SemaphoreType.DMA,
        ],
        compiler_params=pltpu.CompilerParams(use_tc_tiling_on_sc=False,
                                             needs_layout_passes=False),
    )
    def sck(g_hbm, u_hbm, vidx_hbm, pos_hbm, neg_hbm,
            vidx_v, u_v, rows_v, pos_v, neg_v, sem0, sem1):
        cid = lax.axis_index("c")
        sid = lax.axis_index("s")
        wid = sid * _NC + cid
        b0 = wid * BPW
        pltpu.sync_copy(vidx_hbm.at[wid], vidx_v)
        pltpu.sync_copy(u_hbm.at[pl.ds(b0, BPW)], u_v)

        sems = (sem0, sem1)
        iota = lax.broadcasted_iota(jnp.int32, (_L,), 0)

        def rows_dma(c, p, q):
            return pltpu.make_async_copy(
                g_hbm.at[u_v.at[pl.ds(c * CL + q * _L, _L)]],
                rows_v.at[pl.ds(p * CL + q * _L, _L)],
                sems[p])

        def start_chunk(c, p):
            for q in range(GPC):
                rows_dma(c, p, q).start()

        def wait_chunk(c, p):
            for q in range(GPC):
                rows_dma(c, p, q).wait()

        def chunk(c, p):
            @pl.when(c + 1 < NCH)
            def _():
                start_chunk(c + 1, 1 - p)
            wait_chunk(c, p)
            for k in range(GPC):
                g = c * GPC + k
                row_base = p * CL + k * _L + iota
                jrow = g // 8
                jcol = (g % 8) * _L

                def decode(v):
                    w = plsc.load_gather(rows_v, [row_base,
                                                  jnp.bitwise_and(v, H - 1)])
                    amt = lax.shift_right_logical(jnp.bitwise_and(v, H), 5)
                    bits = lax.shift_left(lax.shift_right_logical(w, amt), 16)
                    return plsc.bitcast(bits, jnp.float32)

                neg = None
                pos = None
                for jp in range(JH):
                    wv = vidx_v[jp * 4 + jrow, pl.ds(jcol, _L)]
                    flo = decode(jnp.bitwise_and(wv, 0xFFFF))
                    fhi = decode(lax.shift_right_logical(wv, 16))
                    neg = flo if neg is None else neg + flo
                    if jp < N - JH:
                        neg = neg + fhi
                    elif pos is None:
                        pos = fhi
                    else:
                        pos = pos + fhi
                pos_v[pl.ds(g * _L, _L)] = pos
                neg_v[pl.ds(g * _L, _L)] = neg

        start_chunk(0, 0)

        def lbody(i, carry):
            chunk(2 * i, 0)
            chunk(2 * i + 1, 1)
            return carry

        lax.fori_loop(0, NCH // 2, lbody, 0)
        pltpu.sync_copy(pos_v, pos_hbm.at[pl.ds(b0, BPW)])
        pltpu.sync_copy(neg_v, neg_hbm.at[pl.ds(b0, BPW)])

    return sck(G, u_idx, vidxT)


def _loss(pos2d, neg2d, B):
    def body(p_ref, n_ref, o_ref):
        p = p_ref[...]
        n = n_ref[...]
        t = jax.nn.log_sigmoid(p) + jax.nn.log_sigmoid(-n)
        o_ref[...] = -jnp.sum(t, keepdims=True).reshape(1, 1) / B

    return pl.pallas_call(
        body,
        out_shape=jax.ShapeDtypeStruct((1, 1), jnp.float32),
    )(pos2d, neg2d)


def kernel(u_pos, v_pos, v_neg, batch_size, U_emb, V_emb):
    B = u_pos.shape[0]
    W = v_pos.shape[1]
    N = v_neg.shape[1]
    VOC, D = U_emb.shape

    VP = ((VOC + 127) // 128) * 128
    VTp = jnp.pad(jnp.transpose(V_emb), ((0, 0), (0, VP - VOC)))
    G, vidxT = _prep(jnp.transpose(U_emb), VTp,
                     jnp.transpose(v_pos).astype(jnp.int32),
                     jnp.transpose(v_neg).astype(jnp.int32), B, W, N)
    u_idx = u_pos.reshape(B).astype(jnp.int32)

    pos_s, neg_s = _sc_scores(G, u_idx, vidxT, B, W, N)
    out = _loss(pos_s.reshape(128, B // 128), neg_s.reshape(128, B // 128), B)
    return out[0, 0]
